# baseline (speedup 1.0000x reference)
import functools
import math

import jax
import jax.numpy as jnp
import numpy as np
from jax import lax
from jax.experimental import pallas as pl
from jax.experimental.pallas import tpu as pltpu
from jax.experimental.pallas import tpu_sc as plsc

D_MODEL = 1024
B = 4
S = 2048
NUM_MOD = 3

NC = 2
NS = 16
NW = NC * NS
S_PER_W = S // NW
ROWS = 16
CHUNK = ROWS * D_MODEL
N_CHUNKS = S_PER_W // ROWS
LANES = 16


def _pe_const(d_model=D_MODEL, max_len=S):
    position = np.arange(max_len, dtype=np.float32)[:, None]
    div_term = np.exp(
        np.arange(0, d_model, 2, dtype=np.float32) * (-math.log(10000.0) / d_model))
    pe = np.zeros((max_len, d_model), dtype=np.float32)
    pe[:, 0::2] = np.sin(position * div_term)
    pe[:, 1::2] = np.cos(position * div_term)
    return jnp.asarray(pe.reshape(-1))


_MESH = plsc.VectorSubcoreMesh(core_axis_name="c", subcore_axis_name="s")


@functools.partial(
    pl.kernel,
    mesh=_MESH,
    out_type=[jax.ShapeDtypeStruct((B * S * D_MODEL,), jnp.float32)] * NUM_MOD,
    scratch_types=[
        pltpu.VMEM((S_PER_W * D_MODEL,), jnp.float32),
        pltpu.VMEM((NUM_MOD * D_MODEL,), jnp.float32),
        pltpu.VMEM((CHUNK,), jnp.float32),
    ],
)
def _sc_kernel(xg, xi, xn, mod, pe, og, oi, on, pe_v, mod_v, xbuf):
    wid = lax.axis_index("s") * NC + lax.axis_index("c")
    base_s = wid * S_PER_W

    pltpu.sync_copy(pe.at[pl.ds(base_s * D_MODEL, S_PER_W * D_MODEL)], pe_v)
    pltpu.sync_copy(mod, mod_v)

    for m, (x_hbm, o_hbm) in enumerate(((xg, og), (xi, oi), (xn, on))):
        mbase = m * D_MODEL

        def tloop(t, _, x_hbm=x_hbm, o_hbm=o_hbm, mbase=mbase):
            b = t // N_CHUNKS
            c = t % N_CHUNKS
            off = (b * S + base_s + c * ROWS) * D_MODEL
            pltpu.sync_copy(x_hbm.at[pl.ds(off, CHUNK)], xbuf)

            def dloop(dc, _):
                mv = mod_v[pl.ds(mbase + dc * LANES, LANES)]

                def rloop(r, _):
                    o2 = r * D_MODEL + dc * LANES
                    xbuf[pl.ds(o2, LANES)] = (
                        xbuf[pl.ds(o2, LANES)]
                        + pe_v[pl.ds(c * CHUNK + o2, LANES)]
                        + mv
                    )
                    return 0

                lax.fori_loop(0, ROWS, rloop, 0)
                return 0

            lax.fori_loop(0, D_MODEL // LANES, dloop, 0)
            pltpu.sync_copy(xbuf, o_hbm.at[pl.ds(off, CHUNK)])
            return 0

        lax.fori_loop(0, B * N_CHUNKS, tloop, 0)


def kernel(x_global, x_img, x_nlp, mod_emb):
    pe = _pe_const()
    outs = _sc_kernel(
        x_global.reshape(-1),
        x_img.reshape(-1),
        x_nlp.reshape(-1),
        mod_emb.reshape(-1),
        pe,
    )
    return tuple(o.reshape(B, S, D_MODEL) for o in outs)


# speedup vs baseline: 2.0693x; 2.0693x over previous
import functools
import math

import jax
import jax.numpy as jnp
import numpy as np
from jax import lax
from jax.experimental import pallas as pl
from jax.experimental.pallas import tpu as pltpu
from jax.experimental.pallas import tpu_sc as plsc

D_MODEL = 1024
B = 4
S = 2048
NUM_MOD = 3

NC = 2
NS = 16
NW = NC * NS
S_PER_W = S // NW
ROWS = 16
CHUNK = ROWS * D_MODEL
N_CHUNKS = S_PER_W // ROWS
CH_PER_M = B * N_CHUNKS
LANES = 16
NBUF = 3


def _pe_const(d_model=D_MODEL, max_len=S):
    position = np.arange(max_len, dtype=np.float32)[:, None]
    div_term = np.exp(
        np.arange(0, d_model, 2, dtype=np.float32) * (-math.log(10000.0) / d_model))
    pe = np.zeros((max_len, d_model), dtype=np.float32)
    pe[:, 0::2] = np.sin(position * div_term)
    pe[:, 1::2] = np.cos(position * div_term)
    return jnp.asarray(pe.reshape(-1))


_MESH = plsc.VectorSubcoreMesh(core_axis_name="c", subcore_axis_name="s")


@functools.partial(
    pl.kernel,
    mesh=_MESH,
    out_type=[jax.ShapeDtypeStruct((B * S * D_MODEL,), jnp.float32)] * NUM_MOD,
    scratch_types=(
        [pltpu.VMEM((S_PER_W * D_MODEL,), jnp.float32)]
        + [pltpu.VMEM((NUM_MOD * D_MODEL,), jnp.float32)]
        + [pltpu.VMEM((CHUNK,), jnp.float32)] * NBUF
        + [pltpu.SemaphoreType.DMA] * (2 * NBUF)
    ),
)
def _sc_kernel(xg, xi, xn, mod, pe, og, oi, on,
               pe_v, mod_v, b0, b1, b2, si0, si1, si2, so0, so1, so2):
    bufs = (b0, b1, b2)
    isems = (si0, si1, si2)
    osems = (so0, so1, so2)

    wid = lax.axis_index("s") * NC + lax.axis_index("c")
    base_s = wid * S_PER_W

    pltpu.sync_copy(pe.at[pl.ds(base_s * D_MODEL, S_PER_W * D_MODEL)], pe_v)
    pltpu.sync_copy(mod, mod_v)

    def fold_mod(m):
        def dcloop(dc, _):
            dv = mod_v[pl.ds(m * D_MODEL + dc * LANES, LANES)]
            if m > 0:
                dv = dv - mod_v[pl.ds((m - 1) * D_MODEL + dc * LANES, LANES)]

            @plsc.parallel_loop(0, S_PER_W, unroll=4)
            def sloop(s_i):
                plsc.addupdate(pe_v.at[pl.ds(s_i * D_MODEL + dc * LANES, LANES)], dv)

            return 0

        lax.fori_loop(0, D_MODEL // LANES, dcloop, 0)

    def chunk_off(t):
        b, c = divmod(t, N_CHUNKS)
        return (b * S + base_s + c * ROWS) * D_MODEL, c * CHUNK

    for m, (x_hbm, o_hbm) in enumerate(((xg, og), (xi, oi), (xn, on))):
        fold_mod(m)

        def start_in(t, x_hbm=x_hbm):
            st = t % NBUF
            xo, _ = chunk_off(t)
            pltpu.make_async_copy(
                x_hbm.at[pl.ds(xo, CHUNK)], bufs[st], isems[st]).start()

        def wait_in(t, x_hbm=x_hbm):
            st = t % NBUF
            xo, _ = chunk_off(t)
            pltpu.make_async_copy(
                x_hbm.at[pl.ds(xo, CHUNK)], bufs[st], isems[st]).wait()

        def start_out(t, o_hbm=o_hbm):
            st = t % NBUF
            xo, _ = chunk_off(t)
            pltpu.make_async_copy(
                bufs[st], o_hbm.at[pl.ds(xo, CHUNK)], osems[st]).start()

        def wait_out(t, o_hbm=o_hbm):
            st = t % NBUF
            xo, _ = chunk_off(t)
            pltpu.make_async_copy(
                bufs[st], o_hbm.at[pl.ds(xo, CHUNK)], osems[st]).wait()

        start_in(0)
        for t in range(CH_PER_M):
            if t + 1 < CH_PER_M:
                if t - NBUF + 1 >= 0:
                    wait_out(t - NBUF + 1)
                start_in(t + 1)
            wait_in(t)

            buf = bufs[t % NBUF]
            _, po = chunk_off(t)

            @plsc.parallel_loop(0, CHUNK // LANES, unroll=8)
            def vloop(i, buf=buf, po=po):
                off = i * LANES
                plsc.addupdate(buf.at[pl.ds(off, LANES)],
                               pe_v[pl.ds(po + off, LANES)])

            start_out(t)
        for t in range(CH_PER_M - NBUF, CH_PER_M):
            wait_out(t)


def kernel(x_global, x_img, x_nlp, mod_emb):
    pe = _pe_const()
    outs = _sc_kernel(
        x_global.reshape(-1),
        x_img.reshape(-1),
        x_nlp.reshape(-1),
        mod_emb.reshape(-1),
        pe,
    )
    return tuple(o.reshape(B, S, D_MODEL) for o in outs)


# speedup vs baseline: 5.0453x; 2.4382x over previous
import functools
import math

import jax
import jax.numpy as jnp
import numpy as np
from jax import lax
from jax.experimental import pallas as pl
from jax.experimental.pallas import tpu as pltpu
from jax.experimental.pallas import tpu_sc as plsc

D_MODEL = 1024
B = 4
S = 2048
NUM_MOD = 3

NC = 2
NS = 16
NW = NC * NS
S_PER_W = S // NW
ROWS = 16
N_CHUNKS = S_PER_W // ROWS
CH_PER_M = B * N_CHUNKS
LANES = 16
NBUF = 3


def _pe_const(d_model=D_MODEL, max_len=S):
    position = np.arange(max_len, dtype=np.float32)[:, None]
    div_term = np.exp(
        np.arange(0, d_model, 2, dtype=np.float32) * (-math.log(10000.0) / d_model))
    pe = np.zeros((max_len, d_model), dtype=np.float32)
    pe[:, 0::2] = np.sin(position * div_term)
    pe[:, 1::2] = np.cos(position * div_term)
    return jnp.asarray(pe)


_MESH = plsc.VectorSubcoreMesh(core_axis_name="c", subcore_axis_name="s")


@functools.partial(
    pl.kernel,
    mesh=_MESH,
    out_type=[jax.ShapeDtypeStruct((B, S, D_MODEL), jnp.float32)] * NUM_MOD,
    compiler_params=pltpu.CompilerParams(use_tc_tiling_on_sc=True),
    scratch_types=(
        [pltpu.VMEM((S_PER_W, D_MODEL), jnp.float32)]
        + [pltpu.VMEM((NUM_MOD, D_MODEL), jnp.float32)]
        + [pltpu.VMEM((ROWS, D_MODEL), jnp.float32)] * NBUF
        + [pltpu.SemaphoreType.DMA] * (2 * NBUF)
    ),
)
def _sc_kernel(xg, xi, xn, mod, pe, og, oi, on,
               pe_v, mod_v, b0, b1, b2, si0, si1, si2, so0, so1, so2):
    bufs = (b0, b1, b2)
    isems = (si0, si1, si2)
    osems = (so0, so1, so2)

    wid = lax.axis_index("s") * NC + lax.axis_index("c")
    base_s = wid * S_PER_W

    pltpu.sync_copy(pe.at[pl.ds(base_s, S_PER_W)], pe_v)
    pltpu.sync_copy(mod, mod_v)

    def fold_mod(m):
        def dcloop(dc, _):
            dv = mod_v[m, pl.ds(dc * LANES, LANES)]
            if m > 0:
                dv = dv - mod_v[m - 1, pl.ds(dc * LANES, LANES)]

            @plsc.parallel_loop(0, S_PER_W, unroll=4)
            def sloop(s_i):
                plsc.addupdate(pe_v.at[s_i, pl.ds(dc * LANES, LANES)], dv)

            return 0

        lax.fori_loop(0, D_MODEL // LANES, dcloop, 0)

    def chunk_idx(t):
        b, c = divmod(t, N_CHUNKS)
        return b, base_s + c * ROWS, c * ROWS

    for m, (x_hbm, o_hbm) in enumerate(((xg, og), (xi, oi), (xn, on))):
        fold_mod(m)

        def start_in(t, x_hbm=x_hbm):
            st = t % NBUF
            b, r0, _ = chunk_idx(t)
            pltpu.make_async_copy(
                x_hbm.at[b, pl.ds(r0, ROWS)], bufs[st], isems[st]).start()

        def wait_in(t, x_hbm=x_hbm):
            st = t % NBUF
            b, r0, _ = chunk_idx(t)
            pltpu.make_async_copy(
                x_hbm.at[b, pl.ds(r0, ROWS)], bufs[st], isems[st]).wait()

        def start_out(t, o_hbm=o_hbm):
            st = t % NBUF
            b, r0, _ = chunk_idx(t)
            pltpu.make_async_copy(
                bufs[st], o_hbm.at[b, pl.ds(r0, ROWS)], osems[st]).start()

        def wait_out(t, o_hbm=o_hbm):
            st = t % NBUF
            b, r0, _ = chunk_idx(t)
            pltpu.make_async_copy(
                bufs[st], o_hbm.at[b, pl.ds(r0, ROWS)], osems[st]).wait()

        start_in(0)
        for t in range(CH_PER_M):
            if t + 1 < CH_PER_M:
                if t - NBUF + 1 >= 0:
                    wait_out(t - NBUF + 1)
                start_in(t + 1)
            wait_in(t)

            buf = bufs[t % NBUF]
            _, _, pr0 = chunk_idx(t)

            def rloop(r, _, buf=buf, pr0=pr0):
                @plsc.parallel_loop(0, D_MODEL // LANES, unroll=8)
                def vloop(dc):
                    plsc.addupdate(buf.at[r, pl.ds(dc * LANES, LANES)],
                                   pe_v[pr0 + r, pl.ds(dc * LANES, LANES)])

                return 0

            lax.fori_loop(0, ROWS, rloop, 0)

            start_out(t)
        for t in range(CH_PER_M - NBUF, CH_PER_M):
            wait_out(t)


def kernel(x_global, x_img, x_nlp, mod_emb):
    pe = _pe_const()
    return tuple(_sc_kernel(x_global, x_img, x_nlp, mod_emb, pe))


# speedup vs baseline: 5.2212x; 1.0349x over previous
import functools
import math

import jax
import jax.numpy as jnp
import numpy as np
from jax import lax
from jax.experimental import pallas as pl
from jax.experimental.pallas import tpu as pltpu
from jax.experimental.pallas import tpu_sc as plsc

D_MODEL = 1024
B = 4
S = 2048
NUM_MOD = 3

NC = 2
NS = 16
NW = NC * NS
S_PER_W = S // NW
ROWS = 16
N_CHUNKS = S_PER_W // ROWS
CH_PER_M = B * N_CHUNKS
LANES = 16
NBUF = 3


def _pe_const(d_model=D_MODEL, max_len=S):
    position = np.arange(max_len, dtype=np.float32)[:, None]
    div_term = np.exp(
        np.arange(0, d_model, 2, dtype=np.float32) * (-math.log(10000.0) / d_model))
    pe = np.zeros((max_len, d_model), dtype=np.float32)
    pe[:, 0::2] = np.sin(position * div_term)
    pe[:, 1::2] = np.cos(position * div_term)
    return jnp.asarray(pe)


_MESH = plsc.VectorSubcoreMesh(core_axis_name="c", subcore_axis_name="s")


@functools.partial(
    pl.kernel,
    mesh=_MESH,
    out_type=[jax.ShapeDtypeStruct((B, S, D_MODEL), jnp.float32)] * NUM_MOD,
    compiler_params=pltpu.CompilerParams(use_tc_tiling_on_sc=True),
    scratch_types=(
        [pltpu.VMEM((S_PER_W, D_MODEL), jnp.float32)]
        + [pltpu.VMEM((NUM_MOD, D_MODEL), jnp.float32)]
        + [pltpu.VMEM((ROWS, D_MODEL), jnp.float32)] * NBUF
        + [pltpu.SemaphoreType.DMA] * (2 * NBUF)
    ),
)
def _sc_kernel(xg, xi, xn, mod, pe, og, oi, on,
               pe_v, mod_v, b0, b1, b2, si0, si1, si2, so0, so1, so2):
    bufs = (b0, b1, b2)
    isems = (si0, si1, si2)
    osems = (so0, so1, so2)

    wid = lax.axis_index("s") * NC + lax.axis_index("c")
    base_s = wid * S_PER_W

    pltpu.sync_copy(pe.at[pl.ds(base_s, S_PER_W)], pe_v)
    pltpu.sync_copy(mod, mod_v)

    def fold_mod(m):
        def dcloop(dc, _):
            dv = mod_v[m, pl.ds(dc * LANES, LANES)]
            if m > 0:
                dv = dv - mod_v[m - 1, pl.ds(dc * LANES, LANES)]

            @plsc.parallel_loop(0, S_PER_W, unroll=4)
            def sloop(s_i):
                plsc.addupdate(pe_v.at[s_i, pl.ds(dc * LANES, LANES)], dv)

            return 0

        lax.fori_loop(0, D_MODEL // LANES, dcloop, 0)

    xs = (xg, xi, xn)
    os_ = (og, oi, on)
    TOT = NUM_MOD * CH_PER_M

    def chunk_idx(t):
        m, u = divmod(t, CH_PER_M)
        b, c = divmod(u, N_CHUNKS)
        return m, b, base_s + c * ROWS, c * ROWS

    def start_in(t):
        st = t % NBUF
        m, b, r0, _ = chunk_idx(t)
        pltpu.make_async_copy(
            xs[m].at[b, pl.ds(r0, ROWS)], bufs[st], isems[st]).start()

    def wait_in(t):
        st = t % NBUF
        m, b, r0, _ = chunk_idx(t)
        pltpu.make_async_copy(
            xs[m].at[b, pl.ds(r0, ROWS)], bufs[st], isems[st]).wait()

    def start_out(t):
        st = t % NBUF
        m, b, r0, _ = chunk_idx(t)
        pltpu.make_async_copy(
            bufs[st], os_[m].at[b, pl.ds(r0, ROWS)], osems[st]).start()

    def wait_out(t):
        st = t % NBUF
        m, b, r0, _ = chunk_idx(t)
        pltpu.make_async_copy(
            bufs[st], os_[m].at[b, pl.ds(r0, ROWS)], osems[st]).wait()

    start_in(0)
    for t in range(TOT):
        if t + 1 < TOT:
            if t - NBUF + 1 >= 0:
                wait_out(t - NBUF + 1)
            start_in(t + 1)
        wait_in(t)

        m, _, _, pr0 = chunk_idx(t)
        if t % CH_PER_M == 0:
            fold_mod(m)
        buf = bufs[t % NBUF]

        def rloop(r, _, buf=buf, pr0=pr0):
            @plsc.parallel_loop(0, D_MODEL // LANES, unroll=8)
            def vloop(dc):
                plsc.addupdate(buf.at[r, pl.ds(dc * LANES, LANES)],
                               pe_v[pr0 + r, pl.ds(dc * LANES, LANES)])

            return 0

        lax.fori_loop(0, ROWS, rloop, 0)

        start_out(t)
    for t in range(TOT - NBUF, TOT):
        wait_out(t)


def kernel(x_global, x_img, x_nlp, mod_emb):
    pe = _pe_const()
    return tuple(_sc_kernel(x_global, x_img, x_nlp, mod_emb, pe))


# speedup vs baseline: 6.5711x; 1.2585x over previous
import functools
import math

import jax
import jax.numpy as jnp
import numpy as np
from jax import lax
from jax.experimental import pallas as pl
from jax.experimental.pallas import tpu as pltpu
from jax.experimental.pallas import tpu_sc as plsc

D_MODEL = 1024
B = 4
S = 2048
NUM_MOD = 3

NC = 2
NS = 16
NW = NC * NS
S_PER_W = S // NW
ROWS = 16
N_CHUNKS = S_PER_W // ROWS
CH_PER_M = B * N_CHUNKS
LANES = 16
NBUF = 3


def _pe_const(d_model=D_MODEL, max_len=S):
    position = np.arange(max_len, dtype=np.float32)[:, None]
    div_term = np.exp(
        np.arange(0, d_model, 2, dtype=np.float32) * (-math.log(10000.0) / d_model))
    pe = np.zeros((max_len, d_model), dtype=np.float32)
    pe[:, 0::2] = np.sin(position * div_term)
    pe[:, 1::2] = np.cos(position * div_term)
    return jnp.asarray(pe)


_MESH = plsc.VectorSubcoreMesh(core_axis_name="c", subcore_axis_name="s")


@functools.partial(
    pl.kernel,
    mesh=_MESH,
    out_type=jax.ShapeDtypeStruct((B, S, D_MODEL), jnp.float32),
    compiler_params=pltpu.CompilerParams(use_tc_tiling_on_sc=True),
    scratch_types=(
        [pltpu.VMEM((S_PER_W, D_MODEL), jnp.float32)]
        + [pltpu.VMEM((1, D_MODEL), jnp.float32)]
        + [pltpu.VMEM((ROWS, D_MODEL), jnp.float32)] * NBUF
        + [pltpu.SemaphoreType.DMA] * (2 * NBUF)
    ),
)
def _sc_kernel(x_hbm, mod_hbm, pe, o_hbm,
               pe_v, mod_v, b0, b1, b2, si0, si1, si2, so0, so1, so2):
    bufs = (b0, b1, b2)
    isems = (si0, si1, si2)
    osems = (so0, so1, so2)

    wid = lax.axis_index("s") * NC + lax.axis_index("c")
    base_s = wid * S_PER_W

    pltpu.sync_copy(pe.at[pl.ds(base_s, S_PER_W)], pe_v)
    pltpu.sync_copy(mod_hbm, mod_v)

    def chunk_idx(t):
        b, c = divmod(t, N_CHUNKS)
        return b, base_s + c * ROWS, c * ROWS

    def start_in(t):
        st = t % NBUF
        b, r0, _ = chunk_idx(t)
        pltpu.make_async_copy(
            x_hbm.at[b, pl.ds(r0, ROWS)], bufs[st], isems[st]).start()

    def wait_in(t):
        st = t % NBUF
        b, r0, _ = chunk_idx(t)
        pltpu.make_async_copy(
            x_hbm.at[b, pl.ds(r0, ROWS)], bufs[st], isems[st]).wait()

    def start_out(t):
        st = t % NBUF
        b, r0, _ = chunk_idx(t)
        pltpu.make_async_copy(
            bufs[st], o_hbm.at[b, pl.ds(r0, ROWS)], osems[st]).start()

    def wait_out(t):
        st = t % NBUF
        b, r0, _ = chunk_idx(t)
        pltpu.make_async_copy(
            bufs[st], o_hbm.at[b, pl.ds(r0, ROWS)], osems[st]).wait()

    start_in(0)

    def dcloop(dc, _):
        dv = mod_v[0, pl.ds(dc * LANES, LANES)]

        @plsc.parallel_loop(0, S_PER_W, unroll=4)
        def sloop(s_i):
            plsc.addupdate(pe_v.at[s_i, pl.ds(dc * LANES, LANES)], dv)

        return 0

    lax.fori_loop(0, D_MODEL // LANES, dcloop, 0)

    for t in range(CH_PER_M):
        if t + 1 < CH_PER_M:
            if t - NBUF + 1 >= 0:
                wait_out(t - NBUF + 1)
            start_in(t + 1)
        wait_in(t)

        buf = bufs[t % NBUF]
        _, _, pr0 = chunk_idx(t)

        def rloop(r, _, buf=buf, pr0=pr0):
            @plsc.parallel_loop(0, D_MODEL // LANES, unroll=8)
            def vloop(dc):
                plsc.addupdate(buf.at[r, pl.ds(dc * LANES, LANES)],
                               pe_v[pr0 + r, pl.ds(dc * LANES, LANES)])

            return 0

        lax.fori_loop(0, ROWS, rloop, 0)

        start_out(t)
    for t in range(CH_PER_M - NBUF, CH_PER_M):
        wait_out(t)


TC_BS = 256


def _tc_body(x_ref, pe_ref, mod_ref, o_ref):
    o_ref[...] = x_ref[...] + (pe_ref[...] + mod_ref[...])[None, :, :]


def _tc_call(x, pe2d, modrow):
    return pl.pallas_call(
        _tc_body,
        grid=(S // TC_BS,),
        in_specs=[
            pl.BlockSpec((B, TC_BS, D_MODEL), lambda i: (0, i, 0)),
            pl.BlockSpec((TC_BS, D_MODEL), lambda i: (i, 0)),
            pl.BlockSpec((1, D_MODEL), lambda i: (0, 0)),
        ],
        out_specs=pl.BlockSpec((B, TC_BS, D_MODEL), lambda i: (0, i, 0)),
        out_shape=jax.ShapeDtypeStruct((B, S, D_MODEL), jnp.float32),
    )(x, pe2d, modrow)


def kernel(x_global, x_img, x_nlp, mod_emb):
    pe = _pe_const()
    out_g = _sc_kernel(x_global, mod_emb[0:1], pe)
    out_i = _tc_call(x_img, pe, mod_emb[1:2])
    out_n = _tc_call(x_nlp, pe, mod_emb[2:3])
    return (out_g, out_i, out_n)


# speedup vs baseline: 6.7885x; 1.0331x over previous
import functools
import math

import jax
import jax.numpy as jnp
import numpy as np
from jax import lax
from jax.experimental import pallas as pl
from jax.experimental.pallas import tpu as pltpu
from jax.experimental.pallas import tpu_sc as plsc

D_MODEL = 1024
B = 4
S = 2048
NUM_MOD = 3

NC = 2
NS = 16
NW = NC * NS
S_PER_W = S // NW
ROWS = 16
N_CHUNKS = S_PER_W // ROWS
CH_PER_M = B * N_CHUNKS
LANES = 16
NBUF = 3


def _pe_const(d_model=D_MODEL, max_len=S):
    position = np.arange(max_len, dtype=np.float32)[:, None]
    div_term = np.exp(
        np.arange(0, d_model, 2, dtype=np.float32) * (-math.log(10000.0) / d_model))
    pe = np.zeros((max_len, d_model), dtype=np.float32)
    pe[:, 0::2] = np.sin(position * div_term)
    pe[:, 1::2] = np.cos(position * div_term)
    return jnp.asarray(pe)


_MESH = plsc.VectorSubcoreMesh(core_axis_name="c", subcore_axis_name="s")


@functools.partial(
    pl.kernel,
    mesh=_MESH,
    out_type=jax.ShapeDtypeStruct((B, S, D_MODEL), jnp.float32),
    compiler_params=pltpu.CompilerParams(use_tc_tiling_on_sc=True),
    scratch_types=(
        [pltpu.VMEM((S_PER_W, D_MODEL), jnp.float32)]
        + [pltpu.VMEM((1, D_MODEL), jnp.float32)]
        + [pltpu.VMEM((ROWS, D_MODEL), jnp.float32)] * NBUF
        + [pltpu.SemaphoreType.DMA] * (2 * NBUF)
    ),
)
def _sc_kernel(x_hbm, mod_hbm, pe, o_hbm,
               pe_v, mod_v, b0, b1, b2, si0, si1, si2, so0, so1, so2):
    bufs = (b0, b1, b2)
    isems = (si0, si1, si2)
    osems = (so0, so1, so2)

    wid = lax.axis_index("s") * NC + lax.axis_index("c")
    base_s = wid * S_PER_W

    pltpu.sync_copy(pe.at[pl.ds(base_s, S_PER_W)], pe_v)
    pltpu.sync_copy(mod_hbm, mod_v)

    def chunk_idx(t):
        b, c = divmod(t, N_CHUNKS)
        return b, base_s + c * ROWS, c * ROWS

    def start_in(t):
        st = t % NBUF
        b, r0, _ = chunk_idx(t)
        pltpu.make_async_copy(
            x_hbm.at[b, pl.ds(r0, ROWS)], bufs[st], isems[st]).start()

    def wait_in(t):
        st = t % NBUF
        b, r0, _ = chunk_idx(t)
        pltpu.make_async_copy(
            x_hbm.at[b, pl.ds(r0, ROWS)], bufs[st], isems[st]).wait()

    def start_out(t):
        st = t % NBUF
        b, r0, _ = chunk_idx(t)
        pltpu.make_async_copy(
            bufs[st], o_hbm.at[b, pl.ds(r0, ROWS)], osems[st]).start()

    def wait_out(t):
        st = t % NBUF
        b, r0, _ = chunk_idx(t)
        pltpu.make_async_copy(
            bufs[st], o_hbm.at[b, pl.ds(r0, ROWS)], osems[st]).wait()

    start_in(0)

    def dcloop(dc, _):
        dv = mod_v[0, pl.ds(dc * LANES, LANES)]

        @plsc.parallel_loop(0, S_PER_W, unroll=4)
        def sloop(s_i):
            plsc.addupdate(pe_v.at[s_i, pl.ds(dc * LANES, LANES)], dv)

        return 0

    lax.fori_loop(0, D_MODEL // LANES, dcloop, 0)

    for t in range(CH_PER_M):
        if t + 1 < CH_PER_M:
            if t - NBUF + 1 >= 0:
                wait_out(t - NBUF + 1)
            start_in(t + 1)
        wait_in(t)

        buf = bufs[t % NBUF]
        _, _, pr0 = chunk_idx(t)

        def rloop(r, _, buf=buf, pr0=pr0):
            @plsc.parallel_loop(0, D_MODEL // LANES, unroll=8)
            def vloop(dc):
                plsc.addupdate(buf.at[r, pl.ds(dc * LANES, LANES)],
                               pe_v[pr0 + r, pl.ds(dc * LANES, LANES)])

            return 0

        lax.fori_loop(0, ROWS, rloop, 0)

        start_out(t)
    for t in range(CH_PER_M - NBUF, CH_PER_M):
        wait_out(t)


TC_BS = 256


def _tc_body(xi_ref, xn_ref, pe_ref, mod_ref, oi_ref, on_ref):
    pe_blk = pe_ref[...]
    oi_ref[...] = xi_ref[...] + (pe_blk + mod_ref[1][None, :])[None, :, :]
    on_ref[...] = xn_ref[...] + (pe_blk + mod_ref[2][None, :])[None, :, :]


def _tc_call(x_img, x_nlp, pe2d, mod_emb):
    return pl.pallas_call(
        _tc_body,
        grid=(S // TC_BS,),
        in_specs=[
            pl.BlockSpec((B, TC_BS, D_MODEL), lambda i: (0, i, 0)),
            pl.BlockSpec((B, TC_BS, D_MODEL), lambda i: (0, i, 0)),
            pl.BlockSpec((TC_BS, D_MODEL), lambda i: (i, 0)),
            pl.BlockSpec((NUM_MOD, D_MODEL), lambda i: (0, 0)),
        ],
        out_specs=[
            pl.BlockSpec((B, TC_BS, D_MODEL), lambda i: (0, i, 0)),
            pl.BlockSpec((B, TC_BS, D_MODEL), lambda i: (0, i, 0)),
        ],
        out_shape=[jax.ShapeDtypeStruct((B, S, D_MODEL), jnp.float32)] * 2,
    )(x_img, x_nlp, pe2d, mod_emb)


def kernel(x_global, x_img, x_nlp, mod_emb):
    pe = _pe_const()
    out_g = _sc_kernel(x_global, mod_emb[0:1], pe)
    out_i, out_n = _tc_call(x_img, x_nlp, pe, mod_emb)
    return (out_g, out_i, out_n)
